# baseline (speedup 1.0000x reference)
import functools

import jax
import jax.numpy as jnp
from jax.experimental import pallas as pl
from jax.experimental.pallas import tpu as pltpu

_KBLK = 512
_MBLK = 512


def _mm_body(a_ref, w_ref, at_ref, wt_ref, b_ref, o_ref):
    k = pl.program_id(1)

    @pl.when(k == 0)
    def _init():
        o_ref[...] = (
            jnp.dot(at_ref[...], wt_ref[...], preferred_element_type=jnp.float32)
            + b_ref[...]
        )

    o_ref[...] += jnp.dot(a_ref[...], w_ref[...], preferred_element_type=jnp.float32)


def kernel(inputs, w, b):
    m, kdim = inputs.shape
    n = w.shape[1]
    nsteps = kdim // _KBLK
    rem = kdim - nsteps * _KBLK
    rpad = max(128, ((rem + 127) // 128) * 128)
    a_tail = jnp.pad(inputs[:, nsteps * _KBLK :], ((0, 0), (0, rpad - rem)))
    w_tail = jnp.pad(w[nsteps * _KBLK :], ((0, rpad - rem), (0, 0)))
    b2 = b.reshape(1, n)
    mgrid = m // _MBLK

    out = pl.pallas_call(
        _mm_body,
        grid=(mgrid, nsteps),
        in_specs=[
            pl.BlockSpec((_MBLK, _KBLK), lambda i, k: (i, k)),
            pl.BlockSpec((_KBLK, n), lambda i, k: (k, 0)),
            pl.BlockSpec((_MBLK, rpad), lambda i, k: (i, 0)),
            pl.BlockSpec((rpad, n), lambda i, k: (0, 0)),
            pl.BlockSpec((1, n), lambda i, k: (0, 0)),
        ],
        out_specs=pl.BlockSpec((_MBLK, n), lambda i, k: (i, 0)),
        out_shape=jax.ShapeDtypeStruct((m, n), jnp.float32),
        compiler_params=pltpu.CompilerParams(
            dimension_semantics=("parallel", "arbitrary"),
        ),
    )(inputs, w, a_tail, w_tail, b2)
    return out


# speedup vs baseline: 1.2986x; 1.2986x over previous
import functools

import jax
import jax.numpy as jnp
from jax.experimental import pallas as pl
from jax.experimental.pallas import tpu as pltpu

_KBLK = 2048
_MBLK = 512


def _mm_body(a_ref, w_ref, at_ref, wt_ref, b_ref, o_ref):
    k = pl.program_id(1)

    @pl.when(k == 0)
    def _init():
        o_ref[...] = (
            jnp.dot(
                at_ref[...].astype(jnp.bfloat16),
                wt_ref[...].astype(jnp.bfloat16),
                preferred_element_type=jnp.float32,
            )
            + b_ref[...]
        )

    o_ref[...] += jnp.dot(
        a_ref[...].astype(jnp.bfloat16),
        w_ref[...].astype(jnp.bfloat16),
        preferred_element_type=jnp.float32,
    )


def kernel(inputs, w, b):
    m, kdim = inputs.shape
    n = w.shape[1]
    nsteps = kdim // _KBLK
    rem = kdim - nsteps * _KBLK
    rpad = max(128, ((rem + 127) // 128) * 128)
    a_tail = jnp.pad(inputs[:, nsteps * _KBLK :], ((0, 0), (0, rpad - rem)))
    w_tail = jnp.pad(w[nsteps * _KBLK :], ((0, rpad - rem), (0, 0)))
    b2 = b.reshape(1, n)
    mgrid = m // _MBLK

    out = pl.pallas_call(
        _mm_body,
        grid=(mgrid, nsteps),
        in_specs=[
            pl.BlockSpec((_MBLK, _KBLK), lambda i, k: (i, k)),
            pl.BlockSpec((_KBLK, n), lambda i, k: (k, 0)),
            pl.BlockSpec((_MBLK, rpad), lambda i, k: (i, 0)),
            pl.BlockSpec((rpad, n), lambda i, k: (0, 0)),
            pl.BlockSpec((1, n), lambda i, k: (0, 0)),
        ],
        out_specs=pl.BlockSpec((_MBLK, n), lambda i, k: (i, 0)),
        out_shape=jax.ShapeDtypeStruct((m, n), jnp.float32),
        compiler_params=pltpu.CompilerParams(
            dimension_semantics=("parallel", "arbitrary"),
        ),
    )(inputs, w, a_tail, w_tail, b2)
    return out


# speedup vs baseline: 1.3566x; 1.0446x over previous
import jax
import jax.numpy as jnp
from jax.experimental import pallas as pl
from jax.experimental.pallas import tpu as pltpu

_KBLK = 512
_MBLK = 1024
_S = 4


def _mm_body(*refs):
    a_refs = refs[:_S]
    w_ref, at_ref, wt_ref, b_ref, o_ref = refs[_S:]
    k = pl.program_id(1)

    @pl.when(k == 0)
    def _init():
        o_ref[...] = (
            jnp.dot(
                at_ref[...].astype(jnp.bfloat16),
                wt_ref[...].astype(jnp.bfloat16),
                preferred_element_type=jnp.float32,
            )
            + b_ref[...]
        )

    acc = o_ref[...]
    for s in range(_S):
        acc += jnp.dot(
            a_refs[s][...].astype(jnp.bfloat16),
            w_ref[pl.ds(s * _KBLK, _KBLK), :].astype(jnp.bfloat16),
            preferred_element_type=jnp.float32,
        )
    o_ref[...] = acc


def kernel(inputs, w, b):
    m, kdim = inputs.shape
    n = w.shape[1]
    kstep = _S * _KBLK
    nsteps = kdim // kstep
    rem = kdim - nsteps * kstep
    rpad = max(128, ((rem + 127) // 128) * 128)
    a_tail = jnp.pad(inputs[:, nsteps * kstep :], ((0, 0), (0, rpad - rem)))
    w_tail = jnp.pad(w[nsteps * kstep :], ((0, rpad - rem), (0, 0)))
    b2 = b.reshape(1, n)
    mgrid = m // _MBLK

    a_specs = [
        pl.BlockSpec((_MBLK, _KBLK), lambda i, k, s=s: (i, k * _S + s))
        for s in range(_S)
    ]
    out = pl.pallas_call(
        _mm_body,
        grid=(mgrid, nsteps),
        in_specs=a_specs
        + [
            pl.BlockSpec((kstep, n), lambda i, k: (k, 0)),
            pl.BlockSpec((_MBLK, rpad), lambda i, k: (i, 0)),
            pl.BlockSpec((rpad, n), lambda i, k: (0, 0)),
            pl.BlockSpec((1, n), lambda i, k: (0, 0)),
        ],
        out_specs=pl.BlockSpec((_MBLK, n), lambda i, k: (i, 0)),
        out_shape=jax.ShapeDtypeStruct((m, n), jnp.float32),
        compiler_params=pltpu.CompilerParams(
            dimension_semantics=("parallel", "arbitrary"),
        ),
    )(*([inputs] * _S), w, a_tail, w_tail, b2)
    return out
